# baseline (speedup 1.0000x reference)
import functools

import jax
import jax.numpy as jnp
from jax import lax
from jax.experimental import pallas as pl
from jax.experimental.pallas import tpu as pltpu
from jax.experimental.pallas import tpu_sc as plsc

DIMS = 64
BATCH = 16384
NUM_CORES = 2
NUM_SUBCORES = 16
NUM_WORKERS = NUM_CORES * NUM_SUBCORES
B_PER_W = BATCH // NUM_WORKERS
CHUNK = 128
NCHUNK = B_PER_W // CHUNK


def _pe_lookup_kernel(xy_ref, pe_x_ref, pe_y_ref, out_x_ref, out_y_ref,
                      idx_v, rows_x, rows_y, sem):
    wid = lax.axis_index("s") * NUM_CORES + lax.axis_index("c")
    base = wid * B_PER_W

    pltpu.sync_copy(xy_ref.at[wid], idx_v)

    copies = []
    for j in range(NCHUNK):
        copies.append(pltpu.async_copy(
            pe_x_ref.at[idx_v.at[0, j]],
            rows_x.at[pl.ds(j * CHUNK, CHUNK)], sem))
        copies.append(pltpu.async_copy(
            pe_y_ref.at[idx_v.at[1, j]],
            rows_y.at[pl.ds(j * CHUNK, CHUNK)], sem))
    for c in copies:
        c.wait()

    pltpu.sync_copy(rows_x, out_x_ref.at[pl.ds(base, B_PER_W)])
    pltpu.sync_copy(rows_y, out_y_ref.at[pl.ds(base, B_PER_W)])


@jax.jit
def kernel(xy_tensor, pe_x, pe_y):
    xy = xy_tensor.astype(jnp.int32).reshape(2, NUM_WORKERS, NCHUNK, CHUNK)
    xy = xy.transpose(1, 0, 2, 3)

    mesh = plsc.VectorSubcoreMesh(core_axis_name="c", subcore_axis_name="s")
    run = pl.kernel(
        _pe_lookup_kernel,
        mesh=mesh,
        compiler_params=pltpu.CompilerParams(use_tc_tiling_on_sc=False),
        out_type=(
            jax.ShapeDtypeStruct((BATCH, DIMS), jnp.float32),
            jax.ShapeDtypeStruct((BATCH, DIMS), jnp.float32),
        ),
        scratch_types=[
            pltpu.VMEM((2, NCHUNK, CHUNK), jnp.int32),
            pltpu.VMEM((B_PER_W, DIMS), jnp.float32),
            pltpu.VMEM((B_PER_W, DIMS), jnp.float32),
            pltpu.SemaphoreType.DMA,
        ],
    )
    out_x, out_y = run(xy, pe_x, pe_y)
    return (out_x, out_y)
